# Initial kernel scaffold; baseline (speedup 1.0000x reference)
import jax
import jax.numpy as jnp
from jax.experimental import pallas as pl


def kernel(proposals, cls_scores):
    raise NotImplementedError("write your pallas kernel here")


# speedup vs baseline: 59.7450x; 59.7450x over previous
import jax
import jax.numpy as jnp
from jax import lax
from jax.experimental import pallas as pl

_IOU_THR = 0.5
_B, _N = 4, 5000
_T = 128
_M = 40
_NP = _M * _T


def _suppress_mat(cx1, cy1, cx2, cy2, ca, rx1, ry1, rx2, ry2, ra):
    xi1 = jnp.maximum(cx1, rx1)
    yi1 = jnp.maximum(cy1, ry1)
    xi2 = jnp.minimum(cx2, rx2)
    yi2 = jnp.minimum(cy2, ry2)
    w = jnp.maximum(xi2 - xi1, 0.0)
    h = jnp.maximum(yi2 - yi1, 0.0)
    inter = w * h
    union = ca + ra - inter
    s = (inter >= _IOU_THR * union) & (union > 0.0)
    return s.astype(jnp.float32)


def _nms_body(x1r, y1r, x2r, y2r, outr):
    it = lax.broadcasted_iota(jnp.int32, (_T, _T), 0)
    iu = lax.broadcasted_iota(jnp.int32, (_T, _T), 1)
    tri = (it < iu).astype(jnp.float32)

    outr[0] = jnp.ones((_M, _T), jnp.float32)

    def block_step(i, _):
        rx1 = x1r[0, pl.ds(i, 1), :]
        ry1 = y1r[0, pl.ds(i, 1), :]
        rx2 = x2r[0, pl.ds(i, 1), :]
        ry2 = y2r[0, pl.ds(i, 1), :]
        ra = (rx2 - rx1) * (ry2 - ry1)
        cx1 = jnp.transpose(rx1)
        cy1 = jnp.transpose(ry1)
        cx2 = jnp.transpose(rx2)
        cy2 = jnp.transpose(ry2)
        ca = jnp.transpose(ra)

        s_ii = _suppress_mat(cx1, cy1, cx2, cy2, ca,
                             rx1, ry1, rx2, ry2, ra) * tri
        pre = outr[0, pl.ds(i, 1), :]

        def fix_body(st):
            kb, _ = st
            cnt = jnp.dot(kb, s_ii, preferred_element_type=jnp.float32)
            kb2 = pre * (cnt <= 0.0).astype(jnp.float32)
            return kb2, jnp.any(kb2 != kb)

        kb, _ = lax.while_loop(lambda st: st[1], fix_body, (pre, True))
        outr[0, pl.ds(i, 1), :] = kb

        def cross_body(j, _):
            rjx1 = x1r[0, pl.ds(j, 1), :]
            rjy1 = y1r[0, pl.ds(j, 1), :]
            rjx2 = x2r[0, pl.ds(j, 1), :]
            rjy2 = y2r[0, pl.ds(j, 1), :]
            rja = (rjx2 - rjx1) * (rjy2 - rjy1)
            s_ij = _suppress_mat(cx1, cy1, cx2, cy2, ca,
                                 rjx1, rjy1, rjx2, rjy2, rja)
            cnt = jnp.dot(kb, s_ij, preferred_element_type=jnp.float32)
            outr[0, pl.ds(j, 1), :] *= (cnt <= 0.0).astype(jnp.float32)
            return 0

        return lax.fori_loop(i + 1, _M, cross_body, 0)

    lax.fori_loop(0, _M, block_step, 0)


def _plane_specs():
    spec = pl.BlockSpec((1, _M, _T), lambda b: (b, 0, 0))
    return [spec] * 4


def kernel(proposals, cls_scores):
    obj = cls_scores[:, :, 1]
    order = jnp.argsort(-obj, axis=1)
    sboxes = jnp.take_along_axis(proposals, order[:, :, None], axis=1)

    pad = jnp.zeros((_B, _NP - _N, 4), sboxes.dtype)
    sp = jnp.concatenate([sboxes, pad], axis=1)
    planes = [sp[:, :, k].reshape(_B, _M, _T) for k in range(4)]

    keepf = pl.pallas_call(
        _nms_body,
        grid=(_B,),
        in_specs=_plane_specs(),
        out_specs=pl.BlockSpec((1, _M, _T), lambda b: (b, 0, 0)),
        out_shape=jax.ShapeDtypeStruct((_B, _M, _T), jnp.float32),
    )(*planes)

    keep = keepf.reshape(_B, _NP)[:, :_N] > 0.5
    perm = jnp.argsort(jnp.logical_not(keep), axis=1, stable=True)
    keep_s = jnp.take_along_axis(keep, perm, axis=1)
    out = jnp.take_along_axis(sboxes, perm[:, :, None], axis=1)
    return out * keep_s[:, :, None].astype(sboxes.dtype)


# speedup vs baseline: 95.1460x; 1.5925x over previous
import jax
import jax.numpy as jnp
from jax import lax
from jax.experimental import pallas as pl

_IOU_THR = 0.5
_B, _N = 4, 5000
_T = 128
_M = 40
_CW = 4
_MP = _M + _CW
_NP = _MP * _T


def _suppress_mat(cx1, cy1, cx2, cy2, ca, rx1, ry1, rx2, ry2, ra):
    xi1 = jnp.maximum(cx1, rx1)
    yi1 = jnp.maximum(cy1, ry1)
    xi2 = jnp.minimum(cx2, rx2)
    yi2 = jnp.minimum(cy2, ry2)
    w = jnp.maximum(xi2 - xi1, 0.0)
    h = jnp.maximum(yi2 - yi1, 0.0)
    inter = w * h
    union = ca + ra - inter
    s = (inter >= _IOU_THR * union) & (union > 0.0)
    return s.astype(jnp.float32)


def _nms_body(x1r, y1r, x2r, y2r, outr):
    it = lax.broadcasted_iota(jnp.int32, (_T, _T), 0)
    iu = lax.broadcasted_iota(jnp.int32, (_T, _T), 1)
    tri = (it < iu).astype(jnp.float32)

    outr[0] = jnp.ones((_MP, _T), jnp.float32)

    def block_step(i, _):
        rx1 = x1r[0, pl.ds(i, 1), :]
        ry1 = y1r[0, pl.ds(i, 1), :]
        rx2 = x2r[0, pl.ds(i, 1), :]
        ry2 = y2r[0, pl.ds(i, 1), :]
        ra = (rx2 - rx1) * (ry2 - ry1)
        cx1 = jnp.transpose(rx1)
        cy1 = jnp.transpose(ry1)
        cx2 = jnp.transpose(rx2)
        cy2 = jnp.transpose(ry2)
        ca = (cx2 - cx1) * (cy2 - cy1)

        s_ii = _suppress_mat(cx1, cy1, cx2, cy2, ca,
                             rx1, ry1, rx2, ry2, ra) * tri
        pre = outr[0, pl.ds(i, 1), :]

        def fix_body(st):
            kb, _ = st
            cnt = jnp.dot(kb, s_ii, preferred_element_type=jnp.float32)
            kb2 = pre * (cnt <= 0.0).astype(jnp.float32)
            return kb2, jnp.any(kb2 != kb)

        kb, _ = lax.while_loop(lambda st: st[1], fix_body, (pre, True))
        outr[0, pl.ds(i, 1), :] = kb

        def cross_body(c, _):
            j = i + 1 + c * _CW
            jx1 = x1r[0, pl.ds(j, _CW), :]
            jy1 = y1r[0, pl.ds(j, _CW), :]
            jx2 = x2r[0, pl.ds(j, _CW), :]
            jy2 = y2r[0, pl.ds(j, _CW), :]
            cnts = []
            for k in range(_CW):
                rjx1 = jx1[k:k + 1, :]
                rjy1 = jy1[k:k + 1, :]
                rjx2 = jx2[k:k + 1, :]
                rjy2 = jy2[k:k + 1, :]
                rja = (rjx2 - rjx1) * (rjy2 - rjy1)
                s_ij = _suppress_mat(cx1, cy1, cx2, cy2, ca,
                                     rjx1, rjy1, rjx2, rjy2, rja)
                cnts.append(jnp.dot(kb, s_ij,
                                    preferred_element_type=jnp.float32))
            ok = (jnp.concatenate(cnts, axis=0) <= 0.0).astype(jnp.float32)
            outr[0, pl.ds(j, _CW), :] *= ok
            return 0

        nc = (_M + 2 - i) // _CW
        return lax.fori_loop(0, nc, cross_body, 0)

    lax.fori_loop(0, _M, block_step, 0)


def _plane_specs():
    spec = pl.BlockSpec((1, _MP, _T), lambda b: (b, 0, 0))
    return [spec] * 4


def kernel(proposals, cls_scores):
    obj = cls_scores[:, :, 1]
    order = jnp.argsort(-obj, axis=1)
    sboxes = jnp.take_along_axis(proposals, order[:, :, None], axis=1)

    pad = jnp.zeros((_B, _NP - _N, 4), sboxes.dtype)
    sp = jnp.concatenate([sboxes, pad], axis=1)
    planes = [sp[:, :, k].reshape(_B, _MP, _T) for k in range(4)]

    keepf = pl.pallas_call(
        _nms_body,
        grid=(_B,),
        in_specs=_plane_specs(),
        out_specs=pl.BlockSpec((1, _MP, _T), lambda b: (b, 0, 0)),
        out_shape=jax.ShapeDtypeStruct((_B, _MP, _T), jnp.float32),
    )(*planes)

    keep = keepf.reshape(_B, _NP)[:, :_N] > 0.5
    perm = jnp.argsort(jnp.logical_not(keep), axis=1, stable=True)
    keep_s = jnp.take_along_axis(keep, perm, axis=1)
    out = jnp.take_along_axis(sboxes, perm[:, :, None], axis=1)
    return out * keep_s[:, :, None].astype(sboxes.dtype)


# speedup vs baseline: 105.2228x; 1.1059x over previous
import jax
import jax.numpy as jnp
from jax import lax
from jax.experimental import pallas as pl

_IOU_THR = 0.5
_B, _N = 4, 5000
_T = 128
_M = 40
_CW = 8
_MP = _M + _CW
_NP = _MP * _T


def _suppress_mat(cx1, cy1, cx2, cy2, ca, rx1, ry1, rx2, ry2, ra):
    xi1 = jnp.maximum(cx1, rx1)
    yi1 = jnp.maximum(cy1, ry1)
    xi2 = jnp.minimum(cx2, rx2)
    yi2 = jnp.minimum(cy2, ry2)
    w = jnp.maximum(xi2 - xi1, 0.0)
    h = jnp.maximum(yi2 - yi1, 0.0)
    inter = w * h
    union = ca + ra - inter
    s = (inter >= _IOU_THR * union) & (union > 0.0)
    return s.astype(jnp.float32)


def _nms_body(x1r, y1r, x2r, y2r, outr):
    it = lax.broadcasted_iota(jnp.int32, (_T, _T), 0)
    iu = lax.broadcasted_iota(jnp.int32, (_T, _T), 1)
    tri = (it < iu).astype(jnp.float32)

    outr[0] = jnp.ones((_MP, _T), jnp.float32)

    def block_step(i, _):
        rx1 = x1r[0, pl.ds(i, 1), :]
        ry1 = y1r[0, pl.ds(i, 1), :]
        rx2 = x2r[0, pl.ds(i, 1), :]
        ry2 = y2r[0, pl.ds(i, 1), :]
        ra = (rx2 - rx1) * (ry2 - ry1)
        cx1 = jnp.transpose(rx1)
        cy1 = jnp.transpose(ry1)
        cx2 = jnp.transpose(rx2)
        cy2 = jnp.transpose(ry2)
        ca = (cx2 - cx1) * (cy2 - cy1)

        s_ii = _suppress_mat(cx1, cy1, cx2, cy2, ca,
                             rx1, ry1, rx2, ry2, ra) * tri
        pre = outr[0, pl.ds(i, 1), :]

        def fix_body(st):
            kb, _ = st
            cnt = jnp.dot(kb, s_ii, preferred_element_type=jnp.float32)
            kb2 = pre * (cnt <= 0.0).astype(jnp.float32)
            return kb2, jnp.any(kb2 != kb)

        kb, _ = lax.while_loop(lambda st: st[1], fix_body, (pre, True))
        outr[0, pl.ds(i, 1), :] = kb

        def cross_body(c, _):
            j = i + 1 + c * _CW
            jx1 = x1r[0, pl.ds(j, _CW), :]
            jy1 = y1r[0, pl.ds(j, _CW), :]
            jx2 = x2r[0, pl.ds(j, _CW), :]
            jy2 = y2r[0, pl.ds(j, _CW), :]
            cnts = []
            for k in range(_CW):
                rjx1 = jx1[k:k + 1, :]
                rjy1 = jy1[k:k + 1, :]
                rjx2 = jx2[k:k + 1, :]
                rjy2 = jy2[k:k + 1, :]
                rja = (rjx2 - rjx1) * (rjy2 - rjy1)
                s_ij = _suppress_mat(cx1, cy1, cx2, cy2, ca,
                                     rjx1, rjy1, rjx2, rjy2, rja)
                cnts.append(jnp.dot(kb, s_ij,
                                    preferred_element_type=jnp.float32))
            ok = (jnp.concatenate(cnts, axis=0) <= 0.0).astype(jnp.float32)
            outr[0, pl.ds(j, _CW), :] *= ok
            return 0

        nc = (_M + _CW - 2 - i) // _CW
        return lax.fori_loop(0, nc, cross_body, 0)

    lax.fori_loop(0, _M, block_step, 0)


def _plane_specs():
    spec = pl.BlockSpec((1, _MP, _T), lambda b: (b, 0, 0))
    return [spec] * 4


def kernel(proposals, cls_scores):
    obj = cls_scores[:, :, 1]
    order = jnp.argsort(-obj, axis=1)
    sboxes = jnp.take_along_axis(proposals, order[:, :, None], axis=1)

    pad = jnp.zeros((_B, _NP - _N, 4), sboxes.dtype)
    sp = jnp.concatenate([sboxes, pad], axis=1)
    planes = [sp[:, :, k].reshape(_B, _MP, _T) for k in range(4)]

    keepf = pl.pallas_call(
        _nms_body,
        grid=(_B,),
        in_specs=_plane_specs(),
        out_specs=pl.BlockSpec((1, _MP, _T), lambda b: (b, 0, 0)),
        out_shape=jax.ShapeDtypeStruct((_B, _MP, _T), jnp.float32),
    )(*planes)

    keep = keepf.reshape(_B, _NP)[:, :_N] > 0.5
    perm = jnp.argsort(jnp.logical_not(keep), axis=1, stable=True)
    keep_s = jnp.take_along_axis(keep, perm, axis=1)
    out = jnp.take_along_axis(sboxes, perm[:, :, None], axis=1)
    return out * keep_s[:, :, None].astype(sboxes.dtype)


# speedup vs baseline: 122.8141x; 1.1672x over previous
import jax
import jax.numpy as jnp
from jax import lax
from jax.experimental import pallas as pl

_IOU_THR = 0.5
_B, _N = 4, 5000
_G = 2
_T = 128
_M = 40
_CW = 8
_MP = _M + _CW
_NP = _MP * _T


def _suppress_mat(cx1, cy1, cx2, cy2, ca, rx1, ry1, rx2, ry2, ra):
    xi1 = jnp.maximum(cx1, rx1)
    yi1 = jnp.maximum(cy1, ry1)
    xi2 = jnp.minimum(cx2, rx2)
    yi2 = jnp.minimum(cy2, ry2)
    w = jnp.maximum(xi2 - xi1, 0.0)
    h = jnp.maximum(yi2 - yi1, 0.0)
    inter = w * h
    union = ca + ra - inter
    s = (inter >= _IOU_THR * union) & (union > 0.0)
    return s.astype(jnp.float32)


def _nms_body(x1r, y1r, x2r, y2r, outr):
    it = lax.broadcasted_iota(jnp.int32, (_T, _T), 0)
    iu = lax.broadcasted_iota(jnp.int32, (_T, _T), 1)
    tri = (it < iu).astype(jnp.float32)

    outr[0] = jnp.ones((_G, _MP, _T), jnp.float32)

    def block_step(i, _):
        cols = []
        s_ii = []
        pre = []
        for b in range(_G):
            rx1 = x1r[0, b, pl.ds(i, 1), :]
            ry1 = y1r[0, b, pl.ds(i, 1), :]
            rx2 = x2r[0, b, pl.ds(i, 1), :]
            ry2 = y2r[0, b, pl.ds(i, 1), :]
            ra = (rx2 - rx1) * (ry2 - ry1)
            cx1 = jnp.transpose(rx1)
            cy1 = jnp.transpose(ry1)
            cx2 = jnp.transpose(rx2)
            cy2 = jnp.transpose(ry2)
            ca = (cx2 - cx1) * (cy2 - cy1)
            cols.append((cx1, cy1, cx2, cy2, ca))
            s_ii.append(_suppress_mat(cx1, cy1, cx2, cy2, ca,
                                      rx1, ry1, rx2, ry2, ra) * tri)
            pre.append(outr[0, b, pl.ds(i, 1), :])

        def fix_body(st):
            kbs, _ = st
            kbs2 = []
            changed = None
            for b in range(_G):
                cnt = jnp.dot(kbs[b], s_ii[b],
                              preferred_element_type=jnp.float32)
                kb2 = pre[b] * (cnt <= 0.0).astype(jnp.float32)
                ch = jnp.any(kb2 != kbs[b])
                changed = ch if changed is None else jnp.logical_or(changed, ch)
                kbs2.append(kb2)
            return tuple(kbs2), changed

        kbs, _ = lax.while_loop(lambda st: st[1], fix_body,
                                (tuple(pre), True))
        for b in range(_G):
            outr[0, b, pl.ds(i, 1), :] = kbs[b]

        def cross_body(c, _):
            j = i + 1 + c * _CW
            for b in range(_G):
                cx1, cy1, cx2, cy2, ca = cols[b]
                jx1 = x1r[0, b, pl.ds(j, _CW), :]
                jy1 = y1r[0, b, pl.ds(j, _CW), :]
                jx2 = x2r[0, b, pl.ds(j, _CW), :]
                jy2 = y2r[0, b, pl.ds(j, _CW), :]
                cnts = []
                for k in range(_CW):
                    rjx1 = jx1[k:k + 1, :]
                    rjy1 = jy1[k:k + 1, :]
                    rjx2 = jx2[k:k + 1, :]
                    rjy2 = jy2[k:k + 1, :]
                    rja = (rjx2 - rjx1) * (rjy2 - rjy1)
                    s_ij = _suppress_mat(cx1, cy1, cx2, cy2, ca,
                                         rjx1, rjy1, rjx2, rjy2, rja)
                    cnts.append(jnp.dot(kbs[b], s_ij,
                                        preferred_element_type=jnp.float32))
                ok = (jnp.concatenate(cnts, axis=0) <= 0.0).astype(jnp.float32)
                outr[0, b, pl.ds(j, _CW), :] *= ok
            return 0

        nc = (_M + _CW - 2 - i) // _CW
        return lax.fori_loop(0, nc, cross_body, 0)

    lax.fori_loop(0, _M, block_step, 0)


def _plane_specs():
    spec = pl.BlockSpec((1, _G, _MP, _T), lambda g: (g, 0, 0, 0))
    return [spec] * 4


def kernel(proposals, cls_scores):
    obj = cls_scores[:, :, 1]
    order = jnp.argsort(-obj, axis=1)
    sboxes = jnp.take_along_axis(proposals, order[:, :, None], axis=1)

    pad = jnp.zeros((_B, _NP - _N, 4), sboxes.dtype)
    sp = jnp.concatenate([sboxes, pad], axis=1)
    planes = [sp[:, :, k].reshape(_B // _G, _G, _MP, _T) for k in range(4)]

    keepf = pl.pallas_call(
        _nms_body,
        grid=(_B // _G,),
        in_specs=_plane_specs(),
        out_specs=pl.BlockSpec((1, _G, _MP, _T), lambda g: (g, 0, 0, 0)),
        out_shape=jax.ShapeDtypeStruct((_B // _G, _G, _MP, _T), jnp.float32),
    )(*planes)

    keep = keepf.reshape(_B, _NP)[:, :_N] > 0.5
    perm = jnp.argsort(jnp.logical_not(keep), axis=1, stable=True)
    keep_s = jnp.take_along_axis(keep, perm, axis=1)
    out = jnp.take_along_axis(sboxes, perm[:, :, None], axis=1)
    return out * keep_s[:, :, None].astype(sboxes.dtype)


# speedup vs baseline: 134.5195x; 1.0953x over previous
import jax
import jax.numpy as jnp
from jax import lax
from jax.experimental import pallas as pl

_IOU_THR = 0.5
_B, _N = 4, 5000
_G = 4
_T = 128
_M = 40
_CW = 8
_MP = _M + _CW
_NP = _MP * _T


def _suppress_mat(cx1, cy1, cx2, cy2, ca, rx1, ry1, rx2, ry2, ra):
    xi1 = jnp.maximum(cx1, rx1)
    yi1 = jnp.maximum(cy1, ry1)
    xi2 = jnp.minimum(cx2, rx2)
    yi2 = jnp.minimum(cy2, ry2)
    w = jnp.maximum(xi2 - xi1, 0.0)
    h = jnp.maximum(yi2 - yi1, 0.0)
    inter = w * h
    union = ca + ra - inter
    s = (inter >= _IOU_THR * union) & (union > 0.0)
    return s.astype(jnp.float32)


def _nms_body(x1r, y1r, x2r, y2r, outr):
    it = lax.broadcasted_iota(jnp.int32, (_T, _T), 0)
    iu = lax.broadcasted_iota(jnp.int32, (_T, _T), 1)
    tri = (it < iu).astype(jnp.float32)

    outr[0] = jnp.ones((_G, _MP, _T), jnp.float32)

    def block_step(i, _):
        cols = []
        s_ii = []
        pre = []
        for b in range(_G):
            rx1 = x1r[0, b, pl.ds(i, 1), :]
            ry1 = y1r[0, b, pl.ds(i, 1), :]
            rx2 = x2r[0, b, pl.ds(i, 1), :]
            ry2 = y2r[0, b, pl.ds(i, 1), :]
            ra = (rx2 - rx1) * (ry2 - ry1)
            cx1 = jnp.transpose(rx1)
            cy1 = jnp.transpose(ry1)
            cx2 = jnp.transpose(rx2)
            cy2 = jnp.transpose(ry2)
            ca = (cx2 - cx1) * (cy2 - cy1)
            cols.append((cx1, cy1, cx2, cy2, ca))
            s_ii.append(_suppress_mat(cx1, cy1, cx2, cy2, ca,
                                      rx1, ry1, rx2, ry2, ra) * tri)
            pre.append(outr[0, b, pl.ds(i, 1), :])

        def fix_body(st):
            kbs, _ = st
            kbs2 = []
            changed = None
            for b in range(_G):
                cnt = jnp.dot(kbs[b], s_ii[b],
                              preferred_element_type=jnp.float32)
                kb2 = pre[b] * (cnt <= 0.0).astype(jnp.float32)
                ch = jnp.any(kb2 != kbs[b])
                changed = ch if changed is None else jnp.logical_or(changed, ch)
                kbs2.append(kb2)
            return tuple(kbs2), changed

        kbs, _ = lax.while_loop(lambda st: st[1], fix_body,
                                (tuple(pre), True))
        for b in range(_G):
            outr[0, b, pl.ds(i, 1), :] = kbs[b]

        def cross_body(c, _):
            j = i + 1 + c * _CW
            for b in range(_G):
                cx1, cy1, cx2, cy2, ca = cols[b]
                jx1 = x1r[0, b, pl.ds(j, _CW), :]
                jy1 = y1r[0, b, pl.ds(j, _CW), :]
                jx2 = x2r[0, b, pl.ds(j, _CW), :]
                jy2 = y2r[0, b, pl.ds(j, _CW), :]
                cnts = []
                for k in range(_CW):
                    rjx1 = jx1[k:k + 1, :]
                    rjy1 = jy1[k:k + 1, :]
                    rjx2 = jx2[k:k + 1, :]
                    rjy2 = jy2[k:k + 1, :]
                    rja = (rjx2 - rjx1) * (rjy2 - rjy1)
                    s_ij = _suppress_mat(cx1, cy1, cx2, cy2, ca,
                                         rjx1, rjy1, rjx2, rjy2, rja)
                    cnts.append(jnp.dot(kbs[b], s_ij,
                                        preferred_element_type=jnp.float32))
                ok = (jnp.concatenate(cnts, axis=0) <= 0.0).astype(jnp.float32)
                outr[0, b, pl.ds(j, _CW), :] *= ok
            return 0

        nc = (_M + _CW - 2 - i) // _CW
        return lax.fori_loop(0, nc, cross_body, 0)

    lax.fori_loop(0, _M, block_step, 0)


def _plane_specs():
    spec = pl.BlockSpec((1, _G, _MP, _T), lambda g: (g, 0, 0, 0))
    return [spec] * 4


def kernel(proposals, cls_scores):
    obj = cls_scores[:, :, 1]
    order = jnp.argsort(-obj, axis=1)
    sboxes = jnp.take_along_axis(proposals, order[:, :, None], axis=1)

    pad = jnp.zeros((_B, _NP - _N, 4), sboxes.dtype)
    sp = jnp.concatenate([sboxes, pad], axis=1)
    planes = [sp[:, :, k].reshape(_B // _G, _G, _MP, _T) for k in range(4)]

    keepf = pl.pallas_call(
        _nms_body,
        grid=(_B // _G,),
        in_specs=_plane_specs(),
        out_specs=pl.BlockSpec((1, _G, _MP, _T), lambda g: (g, 0, 0, 0)),
        out_shape=jax.ShapeDtypeStruct((_B // _G, _G, _MP, _T), jnp.float32),
    )(*planes)

    keep = keepf.reshape(_B, _NP)[:, :_N] > 0.5
    perm = jnp.argsort(jnp.logical_not(keep), axis=1, stable=True)
    keep_s = jnp.take_along_axis(keep, perm, axis=1)
    out = jnp.take_along_axis(sboxes, perm[:, :, None], axis=1)
    return out * keep_s[:, :, None].astype(sboxes.dtype)
